# baseline (speedup 1.0000x reference)
import functools

import jax
import jax.numpy as jnp
from jax import lax
from jax.experimental import pallas as pl
from jax.experimental.pallas import tpu as pltpu
from jax.experimental.pallas import tpu_sc as plsc

_K = 8
_NEG = float("-inf")
_D8 = 8
_D4 = 4
_NLEV = _D8 + _D4
_R = 256
_ST = 19456


def _insert_list(levels, v):
    out = []
    for m in levels:
        out.append(jnp.maximum(m, v))
        v = jnp.minimum(m, v)
    return out


def _tc_body(x_ref, o_ref, *, ngrp, c):
    nlev = 2 * _NLEV
    flat = [jnp.full((8, c), _NEG, jnp.float32) for _ in range(nlev)]

    def unflatten(f):
        return [
            (f[0:_D8], f[_D8:_NLEV]),
            (f[_NLEV : _NLEV + _D8], f[_NLEV + _D8 :]),
        ]

    def flatten(gs):
        out = []
        for ms, mn in gs:
            out.extend(ms)
            out.extend(mn)
        return tuple(out)

    def step(i, f):
        gs = unflatten(list(f))
        for p in range(4):
            w1 = x_ref[0, pl.ds((i * 8 + 2 * p) * 8, 8), :]
            w2 = x_ref[0, pl.ds((i * 8 + 2 * p + 1) * 8, 8), :]
            hi = jnp.maximum(w1, w2)
            lo = jnp.minimum(w1, w2)
            ms, mn = gs[p % 2]
            gs[p % 2] = (_insert_list(ms, hi), _insert_list(mn, lo))
        return flatten(gs)

    flat = jax.lax.fori_loop(0, ngrp // 8, step, tuple(flat))

    ms = list(flat)
    sub = jax.lax.broadcasted_iota(jnp.int32, (8, c), 0)
    big = jnp.int32(1 << 20)
    rows = []
    for _ in range(_K):
        cur = functools.reduce(jnp.maximum, ms)
        colmax = jnp.max(cur, axis=0, keepdims=True)
        rows.append(colmax)
        bc = jnp.broadcast_to(colmax, (8, c))
        idxs = [
            jnp.where(ms[i] == bc, i * 8 + sub, big) for i in range(nlev)
        ]
        mini = functools.reduce(jnp.minimum, idxs)
        mcol = jnp.min(mini, axis=0, keepdims=True)
        mbc = jnp.broadcast_to(mcol, (8, c))
        ms = [
            jnp.where((ms[i] == bc) & ((i * 8 + sub) == mbc), _NEG, ms[i])
            for i in range(nlev)
        ]
    o_ref[0] = jnp.concatenate(rows, axis=0)


def _tc_stage(x):
    b, s, c = x.shape
    return pl.pallas_call(
        functools.partial(_tc_body, ngrp=_ST // 8, c=c),
        grid=(b,),
        in_specs=[pl.BlockSpec((1, _ST, c), lambda bi: (bi, 0, 0))],
        out_specs=pl.BlockSpec((1, _K, c), lambda bi: (bi, 0, 0)),
        out_shape=jax.ShapeDtypeStruct((b, _K, c), jnp.float32),
        compiler_params=pltpu.CompilerParams(
            dimension_semantics=("arbitrary",)
        ),
    )(x)


def _sc_stage(x):
    b, s, c = x.shape
    slab = s // 2
    nchunk = slab // _R
    mesh = plsc.VectorSubcoreMesh(core_axis_name="c", subcore_axis_name="s")

    @functools.partial(
        pl.kernel,
        mesh=mesh,
        out_type=jax.ShapeDtypeStruct((b, 2, _NLEV, c), jnp.float32),
        scratch_types=[
            pltpu.VMEM((2, _R, c), jnp.float32),
            pltpu.VMEM((_NLEV, c), jnp.float32),
            pltpu.SemaphoreType.DMA,
            pltpu.SemaphoreType.DMA,
        ],
    )
    def k(x_hbm, out_hbm, buf, cand, sem0, sem1):
        wid = lax.axis_index("s") * 2 + lax.axis_index("c")
        bi = wid // 2
        hf = wid % 2
        base = hf * slab
        sems = (sem0, sem1)

        for l in range(_NLEV):
            for g in range(4):
                cand[l, pl.ds(g * 16, 16)] = jnp.full((16,), _NEG, jnp.float32)

        pltpu.make_async_copy(
            x_hbm.at[bi, pl.ds(base, _R)], buf.at[0], sems[0]
        ).start()

        def insert(f, v, off, depth):
            for l in range(depth):
                m = f[off + l]
                f[off + l] = jnp.maximum(m, v)
                v = jnp.minimum(m, v)
            return f

        def chunk_pair(ci2, _unused):
            for sub in range(2):
                ci = ci2 * 2 + sub

                @pl.when(ci + 1 < nchunk)
                def _():
                    pltpu.make_async_copy(
                        x_hbm.at[bi, pl.ds(base + (ci + 1) * _R, _R)],
                        buf.at[1 - sub],
                        sems[1 - sub],
                    ).start()

                pltpu.make_async_copy(
                    x_hbm.at[bi, pl.ds(base + ci * _R, _R)],
                    buf.at[sub],
                    sems[sub],
                ).wait()

                for ps in range(2):
                    ga, gb = ps, ps + 2
                    init = []
                    for g in (ga, gb):
                        for l in range(_NLEV):
                            init.append(cand[l, pl.ds(g * 16, 16)])

                    def row_body(r, f, sub=sub, ga=ga, gb=gb):
                        f = list(f)
                        for kk, g in enumerate((ga, gb)):
                            va = buf[sub, 2 * r, pl.ds(g * 16, 16)]
                            vb = buf[sub, 2 * r + 1, pl.ds(g * 16, 16)]
                            h = jnp.maximum(va, vb)
                            lo = jnp.minimum(va, vb)
                            f = insert(f, h, kk * _NLEV, _D8)
                            f = insert(f, lo, kk * _NLEV + _D8, _D4)
                        return tuple(f)

                    fin = lax.fori_loop(0, _R // 2, row_body, tuple(init))
                    for kk, g in enumerate((ga, gb)):
                        for l in range(_NLEV):
                            cand[l, pl.ds(g * 16, 16)] = fin[kk * _NLEV + l]
            return 0

        lax.fori_loop(0, nchunk // 2, chunk_pair, 0)
        pltpu.sync_copy(cand, out_hbm.at[bi, hf])

    return k(x)


def _merge_body(t_ref, c_ref, o_ref):
    nlev = _K + 2 * _NLEV
    ms = [t_ref[0, pl.ds(i, 1), :] for i in range(_K)]
    ms += [
        c_ref[0, i // _NLEV, pl.ds(i % _NLEV, 1), :]
        for i in range(2 * _NLEV)
    ]
    big = jnp.int32(1 << 20)
    rows = []
    for _ in range(_K):
        cur = functools.reduce(jnp.maximum, ms)
        rows.append(cur)
        idxs = [
            jnp.where(ms[i] == cur, jnp.int32(i), big) for i in range(nlev)
        ]
        mini = functools.reduce(jnp.minimum, idxs)
        ms = [
            jnp.where((ms[i] == cur) & (mini == jnp.int32(i)), _NEG, ms[i])
            for i in range(nlev)
        ]
    o_ref[0] = jnp.concatenate(rows, axis=0)


def _merge(tc_out, cand):
    b = cand.shape[0]
    c = cand.shape[-1]
    return pl.pallas_call(
        _merge_body,
        grid=(b,),
        in_specs=[
            pl.BlockSpec((1, _K, c), lambda bi: (bi, 0, 0)),
            pl.BlockSpec((1, 2, _NLEV, c), lambda bi: (bi, 0, 0, 0)),
        ],
        out_specs=pl.BlockSpec((1, _K, c), lambda bi: (bi, 0, 0)),
        out_shape=jax.ShapeDtypeStruct((b, _K, c), jnp.float32),
    )(tc_out, cand)


def kernel(inputs):
    b, s, c = inputs.shape
    cand = _sc_stage(inputs[:, _ST:, :])
    tc_out = _tc_stage(inputs)
    out = _merge(tc_out, cand)
    return out.transpose(0, 2, 1).reshape(b, c * _K)


# speedup vs baseline: 1.2834x; 1.2834x over previous
import functools

import jax
import jax.numpy as jnp
from jax import lax
from jax.experimental import pallas as pl
from jax.experimental.pallas import tpu as pltpu
from jax.experimental.pallas import tpu_sc as plsc

_K = 8
_NEG = float("-inf")
_D8 = 8
_D4 = 4
_NLEV = _D8 + _D4
_R = 256
_ST = 15360


def _insert_list(levels, v):
    out = []
    for m in levels:
        out.append(jnp.maximum(m, v))
        v = jnp.minimum(m, v)
    return out


def _tc_body(x_ref, o_ref, *, ngrp, c):
    nlev = 2 * _NLEV
    flat = [jnp.full((8, c), _NEG, jnp.float32) for _ in range(nlev)]

    def unflatten(f):
        return [
            (f[0:_D8], f[_D8:_NLEV]),
            (f[_NLEV : _NLEV + _D8], f[_NLEV + _D8 :]),
        ]

    def flatten(gs):
        out = []
        for ms, mn in gs:
            out.extend(ms)
            out.extend(mn)
        return tuple(out)

    def step(i, f):
        gs = unflatten(list(f))
        for p in range(4):
            w1 = x_ref[0, pl.ds((i * 8 + 2 * p) * 8, 8), :]
            w2 = x_ref[0, pl.ds((i * 8 + 2 * p + 1) * 8, 8), :]
            hi = jnp.maximum(w1, w2)
            lo = jnp.minimum(w1, w2)
            ms, mn = gs[p % 2]
            gs[p % 2] = (_insert_list(ms, hi), _insert_list(mn, lo))
        return flatten(gs)

    flat = jax.lax.fori_loop(0, ngrp // 8, step, tuple(flat))

    ms = list(flat)
    sub = jax.lax.broadcasted_iota(jnp.int32, (8, c), 0)
    big = jnp.int32(1 << 20)
    rows = []
    for _ in range(_K):
        cur = functools.reduce(jnp.maximum, ms)
        colmax = jnp.max(cur, axis=0, keepdims=True)
        rows.append(colmax)
        bc = jnp.broadcast_to(colmax, (8, c))
        idxs = [
            jnp.where(ms[i] == bc, i * 8 + sub, big) for i in range(nlev)
        ]
        mini = functools.reduce(jnp.minimum, idxs)
        mcol = jnp.min(mini, axis=0, keepdims=True)
        mbc = jnp.broadcast_to(mcol, (8, c))
        ms = [
            jnp.where((ms[i] == bc) & ((i * 8 + sub) == mbc), _NEG, ms[i])
            for i in range(nlev)
        ]
    o_ref[0] = jnp.concatenate(rows, axis=0)


def _tc_stage(x):
    b, s, c = x.shape
    return pl.pallas_call(
        functools.partial(_tc_body, ngrp=_ST // 8, c=c),
        grid=(b,),
        in_specs=[pl.BlockSpec((1, _ST, c), lambda bi: (bi, 0, 0))],
        out_specs=pl.BlockSpec((1, _K, c), lambda bi: (bi, 0, 0)),
        out_shape=jax.ShapeDtypeStruct((b, _K, c), jnp.float32),
        compiler_params=pltpu.CompilerParams(
            dimension_semantics=("arbitrary",)
        ),
    )(x)


def _sc_stage(x):
    b, s, c = x.shape
    slab = (s - _ST) // 2
    nchunk = slab // _R
    mesh = plsc.VectorSubcoreMesh(core_axis_name="c", subcore_axis_name="s")

    @functools.partial(
        pl.kernel,
        mesh=mesh,
        out_type=jax.ShapeDtypeStruct((b, 2, _NLEV, c), jnp.float32),
        scratch_types=[
            pltpu.VMEM((2, _R, c), jnp.float32),
            pltpu.VMEM((_NLEV, c), jnp.float32),
            pltpu.SemaphoreType.DMA,
            pltpu.SemaphoreType.DMA,
        ],
    )
    def k(x_hbm, out_hbm, buf, cand, sem0, sem1):
        wid = lax.axis_index("s") * 2 + lax.axis_index("c")
        bi = wid // 2
        hf = wid % 2
        base = _ST + hf * slab
        sems = (sem0, sem1)

        for l in range(_NLEV):
            for g in range(4):
                cand[l, pl.ds(g * 16, 16)] = jnp.full((16,), _NEG, jnp.float32)

        pltpu.make_async_copy(
            x_hbm.at[bi, pl.ds(base, _R)], buf.at[0], sems[0]
        ).start()

        def insert(f, v, off, depth):
            for l in range(depth):
                m = f[off + l]
                f[off + l] = jnp.maximum(m, v)
                v = jnp.minimum(m, v)
            return f

        def chunk_pair(ci2, _unused):
            for sub in range(2):
                ci = ci2 * 2 + sub

                @pl.when(ci + 1 < nchunk)
                def _():
                    pltpu.make_async_copy(
                        x_hbm.at[bi, pl.ds(base + (ci + 1) * _R, _R)],
                        buf.at[1 - sub],
                        sems[1 - sub],
                    ).start()

                pltpu.make_async_copy(
                    x_hbm.at[bi, pl.ds(base + ci * _R, _R)],
                    buf.at[sub],
                    sems[sub],
                ).wait()

                for ps in range(2):
                    ga, gb = ps, ps + 2
                    init = []
                    for g in (ga, gb):
                        for l in range(_NLEV):
                            init.append(cand[l, pl.ds(g * 16, 16)])

                    def row_body(r, f, sub=sub, ga=ga, gb=gb):
                        f = list(f)
                        for kk, g in enumerate((ga, gb)):
                            va = buf[sub, 2 * r, pl.ds(g * 16, 16)]
                            vb = buf[sub, 2 * r + 1, pl.ds(g * 16, 16)]
                            h = jnp.maximum(va, vb)
                            lo = jnp.minimum(va, vb)
                            f = insert(f, h, kk * _NLEV, _D8)
                            f = insert(f, lo, kk * _NLEV + _D8, _D4)
                        return tuple(f)

                    fin = lax.fori_loop(0, _R // 2, row_body, tuple(init))
                    for kk, g in enumerate((ga, gb)):
                        for l in range(_NLEV):
                            cand[l, pl.ds(g * 16, 16)] = fin[kk * _NLEV + l]
            return 0

        lax.fori_loop(0, nchunk // 2, chunk_pair, 0)
        pltpu.sync_copy(cand, out_hbm.at[bi, hf])

    return k(x)


def _merge_body(t_ref, c_ref, o_ref):
    nlev = _K + 2 * _NLEV
    ms = [t_ref[0, pl.ds(i, 1), :] for i in range(_K)]
    ms += [
        c_ref[0, i // _NLEV, pl.ds(i % _NLEV, 1), :]
        for i in range(2 * _NLEV)
    ]
    big = jnp.int32(1 << 20)
    rows = []
    for _ in range(_K):
        cur = functools.reduce(jnp.maximum, ms)
        rows.append(cur)
        idxs = [
            jnp.where(ms[i] == cur, jnp.int32(i), big) for i in range(nlev)
        ]
        mini = functools.reduce(jnp.minimum, idxs)
        ms = [
            jnp.where((ms[i] == cur) & (mini == jnp.int32(i)), _NEG, ms[i])
            for i in range(nlev)
        ]
    o_ref[0] = jnp.concatenate(rows, axis=0)


def _merge(tc_out, cand):
    b = cand.shape[0]
    c = cand.shape[-1]
    return pl.pallas_call(
        _merge_body,
        grid=(b,),
        in_specs=[
            pl.BlockSpec((1, _K, c), lambda bi: (bi, 0, 0)),
            pl.BlockSpec((1, 2, _NLEV, c), lambda bi: (bi, 0, 0, 0)),
        ],
        out_specs=pl.BlockSpec((1, _K, c), lambda bi: (bi, 0, 0)),
        out_shape=jax.ShapeDtypeStruct((b, _K, c), jnp.float32),
    )(tc_out, cand)


def kernel(inputs):
    b, s, c = inputs.shape
    cand = _sc_stage(inputs)
    tc_out = _tc_stage(inputs)
    out = _merge(tc_out, cand)
    return out.transpose(0, 2, 1).reshape(b, c * _K)
